# baseline (speedup 1.0000x reference)
import functools

import jax
import jax.numpy as jnp
from jax import lax
from jax.experimental import pallas as pl
from jax.experimental.pallas import tpu as pltpu
from jax.experimental.pallas import tpu_sc as plsc

N_REAL = 50000
N_PAD = 50176
E_REAL = 1600000
E_PAD = 1638400
N_CORES = 2
N_SUBCORES = 16
E_PER_CORE = E_PAD // N_CORES
E_PER_TILE = E_PER_CORE // N_SUBCORES
BATCH = 400
N_IDXROWS = E_PER_TILE // BATCH
NBUF = 3
DEPTH = 2
CHUNK = 8
STRIPE = N_PAD // N_SUBCORES
ROWS_BLK = 1024
GRID_ROWS = N_PAD // ROWS_BLK
NCH1 = 3
NCH2 = 7


def _make_prop(fc, nch):
    mesh = plsc.VectorSubcoreMesh(core_axis_name="c", subcore_axis_name="s")

    @functools.partial(
        pl.kernel,
        out_type=jax.ShapeDtypeStruct((nch, N_CORES, N_PAD, fc), jnp.float32),
        mesh=mesh,
        scratch_types=[
            pltpu.VMEM((CHUNK, BATCH), jnp.int32),
            pltpu.VMEM((CHUNK, BATCH), jnp.int32),
            [pltpu.VMEM((BATCH, fc), jnp.float32) for _ in range(NBUF)],
            pltpu.VMEM_SHARED((N_PAD, fc), jnp.float32),
            pltpu.VMEM_SHARED((N_PAD, fc), jnp.float32),
            [pltpu.SemaphoreType.DMA for _ in range(NBUF)],
            [pltpu.SemaphoreType.DMA for _ in range(NBUF)],
        ],
        compiler_params=pltpu.CompilerParams(use_tc_tiling_on_sc=False),
    )
    def prop(table, srcs2, dsts2, out, sbuf, dbuf, rows, acc, tbl, gsems, ssems):
        c = lax.axis_index("c")
        s = lax.axis_index("s")
        r0 = s * STRIPE
        rb0 = c * (E_PER_CORE // BATCH) + s * N_IDXROWS

        def chunk_body(k, carry):
            rb = rb0 + k * CHUNK
            pltpu.sync_copy(srcs2.at[pl.ds(rb, CHUNK)], sbuf)
            pltpu.sync_copy(dsts2.at[pl.ds(rb, CHUNK)], dbuf)
            for g in range(DEPTH):
                pltpu.async_copy(tbl.at[sbuf.at[g]], rows[g], gsems[g])
            for g in range(CHUNK):
                b = g % NBUF
                pltpu.make_async_copy(tbl.at[sbuf.at[g]], rows[b], gsems[b]).wait()
                pltpu.async_copy(rows[b], acc.at[dbuf.at[g]], ssems[b], add=True)
                ng = g + DEPTH
                if ng < CHUNK:
                    bn = ng % NBUF
                    if ng >= NBUF:
                        pltpu.make_async_copy(
                            rows[bn], acc.at[dbuf.at[ng - NBUF]], ssems[bn]).wait()
                    pltpu.async_copy(tbl.at[sbuf.at[ng]], rows[bn], gsems[bn])
            for g in range(CHUNK - NBUF, CHUNK):
                b = g % NBUF
                pltpu.make_async_copy(rows[b], acc.at[dbuf.at[g]], ssems[b]).wait()
            return carry

        for cc in range(nch):
            pltpu.sync_copy(table.at[cc, pl.ds(r0, STRIPE)],
                            acc.at[pl.ds(r0, STRIPE)])
            pltpu.sync_copy(table.at[cc, pl.ds(r0, STRIPE)],
                            tbl.at[pl.ds(r0, STRIPE)])
            plsc.subcore_barrier()
            lax.fori_loop(0, N_IDXROWS // CHUNK, chunk_body, 0)
            plsc.subcore_barrier()
            pltpu.sync_copy(acc.at[pl.ds(r0, STRIPE)],
                            out.at[cc, c, pl.ds(r0, STRIPE)])

    return prop


_prop16 = _make_prop(16, 1)
_prop1 = _make_prop(1, 1)


def _blk(*shape):
    return pl.BlockSpec(shape, lambda i: (0,) * (len(shape) - 2) + (i, 0))


def _full(*shape):
    return pl.BlockSpec(shape, lambda i: (0,) * len(shape))


def _stage1_body(degp, xpad, dinv_o, *xs_o):
    deg = degp[0, 0] + degp[0, 1] - 1.0
    row = (pl.program_id(0) * ROWS_BLK
           + lax.broadcasted_iota(jnp.int32, (ROWS_BLK, 1), 0))
    dinv = jnp.where(row < N_REAL, lax.rsqrt(deg), 0.0)
    dinv_o[...] = dinv
    xs = dinv * xpad[...]
    for i in range(NCH1):
        xs_o[i][...] = xs[:, 16 * i:16 * (i + 1)]


def _stage1(degp, xpad):
    return pl.pallas_call(
        _stage1_body,
        grid=(GRID_ROWS,),
        in_specs=[_blk(1, 2, ROWS_BLK, 1), _blk(ROWS_BLK, 48)],
        out_specs=(_blk(ROWS_BLK, 1),) + (_blk(ROWS_BLK, 16),) * NCH1,
        out_shape=(jax.ShapeDtypeStruct((N_PAD, 1), jnp.float32),)
        + (jax.ShapeDtypeStruct((N_PAD, 16), jnp.float32),) * NCH1,
    )(degp, xpad)


def _stage2_body(*refs):
    z1 = refs[:NCH1]
    xs = refs[NCH1:2 * NCH1]
    dinv, w1, b1, w2p = refs[2 * NCH1:2 * NCH1 + 4]
    t2_o = refs[2 * NCH1 + 4:]
    d = dinv[...]
    h1 = b1[...]
    for i in range(NCH1):
        zn = d * (z1[i][0, 0] + z1[i][0, 1] - xs[i][...])
        h1 = h1 + jnp.dot(zn, w1[16 * i:16 * (i + 1), :],
                          preferred_element_type=jnp.float32)
    h1 = jnp.maximum(h1, 0.0)
    t2 = d * jnp.dot(h1, w2p[...], preferred_element_type=jnp.float32)
    for i in range(NCH2):
        t2_o[i][...] = t2[:, 16 * i:16 * (i + 1)]


def _stage2(z1s, xss, dinv, w1, b1r, w2p):
    return pl.pallas_call(
        _stage2_body,
        grid=(GRID_ROWS,),
        in_specs=[_blk(1, 2, ROWS_BLK, 16)] * NCH1 + [_blk(ROWS_BLK, 16)] * NCH1
        + [_blk(ROWS_BLK, 1), _full(48, 300), _full(1, 300), _full(300, 112)],
        out_specs=(_blk(ROWS_BLK, 16),) * NCH2,
        out_shape=(jax.ShapeDtypeStruct((N_PAD, 16), jnp.float32),) * NCH2,
    )(*z1s, *xss, dinv, w1, b1r, w2p)


def _stage3_body(*refs):
    z2 = refs[:NCH2]
    t2 = refs[NCH2:2 * NCH2]
    dinv, b2p, w3p = refs[2 * NCH2:2 * NCH2 + 3]
    t3_o = refs[-1]
    d = dinv[...]
    acc = jnp.zeros((ROWS_BLK, 1), jnp.float32)
    for i in range(NCH2):
        h = jnp.maximum(
            d * (z2[i][0, 0] + z2[i][0, 1] - t2[i][...]) + b2p[:, 16 * i:16 * (i + 1)],
            0.0)
        acc = acc + jnp.dot(h, w3p[16 * i:16 * (i + 1), :],
                            preferred_element_type=jnp.float32)
    t3_o[...] = d * acc


def _stage3(z2s, t2s, dinv, b2p, w3p):
    return pl.pallas_call(
        _stage3_body,
        grid=(GRID_ROWS,),
        in_specs=[_blk(1, 2, ROWS_BLK, 16)] * NCH2 + [_blk(ROWS_BLK, 16)] * NCH2
        + [_blk(ROWS_BLK, 1), _full(1, 112), _full(112, 1)],
        out_specs=_blk(ROWS_BLK, 1),
        out_shape=jax.ShapeDtypeStruct((N_PAD, 1), jnp.float32),
    )(*z2s, *t2s, dinv, b2p, w3p)


def _stage4_body(z3, t3, dinv, b3, out_o):
    out_o[...] = dinv[...] * (z3[0, 0] + z3[0, 1] - t3[...]) + b3[0, 0]


def _stage4(z3, t3, dinv, b3):
    return pl.pallas_call(
        _stage4_body,
        grid=(GRID_ROWS,),
        in_specs=[_blk(1, 2, ROWS_BLK, 1), _blk(ROWS_BLK, 1), _blk(ROWS_BLK, 1),
                  _full(1, 1)],
        out_specs=_blk(ROWS_BLK, 1),
        out_shape=jax.ShapeDtypeStruct((N_PAD, 1), jnp.float32),
    )(z3, t3, dinv, b3)


def kernel(x, edge_index, W1, b1, W2, b2, W3, b3):
    pad = jnp.full((E_PAD - E_REAL,), N_REAL, dtype=jnp.int32)
    srcs = jnp.concatenate([edge_index[0], pad]).reshape(E_PAD // BATCH, BATCH)
    dsts = jnp.concatenate([edge_index[1], pad]).reshape(E_PAD // BATCH, BATCH)
    xpad = jnp.pad(x, ((0, N_PAD - N_REAL), (0, 0)))
    ones = jnp.ones((N_PAD, 1), jnp.float32)
    b1r = b1.reshape(1, 300)
    w2p = jnp.pad(W2, ((0, 0), (0, 12)))
    b2p = jnp.pad(b2, (0, 12)).reshape(1, 112)
    w3p = jnp.pad(W3, ((0, 12), (0, 0)))
    b3r = b3.reshape(1, 1)

    degp = _prop1(ones[None], srcs, dsts)
    s1 = _stage1(degp, xpad)
    dinv, xss = s1[0], s1[1:]
    z1s = [_prop16(t[None], srcs, dsts) for t in xss]
    t2s = _stage2(z1s, xss, dinv, W1, b1r, w2p)
    z2s = [_prop16(t[None], srcs, dsts) for t in t2s]
    t3 = _stage3(z2s, t2s, dinv, b2p, w3p)
    z3 = _prop1(t3[None], srcs, dsts)
    out = _stage4(z3, t3, dinv, b3r)
    return out[:N_REAL, 0]


# speedup vs baseline: 1.0855x; 1.0855x over previous
import functools

import jax
import jax.numpy as jnp
from jax import lax
from jax.experimental import pallas as pl
from jax.experimental.pallas import tpu as pltpu
from jax.experimental.pallas import tpu_sc as plsc

N_REAL = 50000
N_PAD = 50176
E_REAL = 1600000
E_PAD = 1638400
N_CORES = 2
N_SUBCORES = 16
E_PER_CORE = E_PAD // N_CORES
E_PER_TILE = E_PER_CORE // N_SUBCORES
BATCH = 320
N_IDXROWS = E_PER_TILE // BATCH
NBUF = 3
DEPTH = 2
CHUNK = 10
STRIPE = N_PAD // N_SUBCORES
ROWS_BLK = 1024
GRID_ROWS = N_PAD // ROWS_BLK
NCH1 = 3
NCH2 = 7


def _make_prop(fc, nch):
    mesh = plsc.VectorSubcoreMesh(core_axis_name="c", subcore_axis_name="s")

    @functools.partial(
        pl.kernel,
        out_type=jax.ShapeDtypeStruct((nch, N_CORES, N_PAD, fc), jnp.float32),
        mesh=mesh,
        scratch_types=[
            [pltpu.VMEM((CHUNK, BATCH), jnp.int32) for _ in range(2)],
            [pltpu.VMEM((CHUNK, BATCH), jnp.int32) for _ in range(2)],
            [pltpu.VMEM((BATCH, fc), jnp.float32) for _ in range(NBUF)],
            pltpu.VMEM_SHARED((N_PAD, fc), jnp.float32),
            pltpu.VMEM_SHARED((N_PAD, fc), jnp.float32),
            [pltpu.SemaphoreType.DMA for _ in range(NBUF)],
            [pltpu.SemaphoreType.DMA for _ in range(NBUF)],
            [pltpu.SemaphoreType.DMA for _ in range(2)],
        ],
        compiler_params=pltpu.CompilerParams(use_tc_tiling_on_sc=False),
    )
    def prop(table, srcs2, dsts2, out, sbufs, dbufs, rows, acc, tbl,
             gsems, ssems, isems):
        c = lax.axis_index("c")
        s = lax.axis_index("s")
        r0 = s * STRIPE
        rb0 = c * (E_PER_CORE // BATCH) + s * N_IDXROWS
        n_ch = N_IDXROWS // CHUNK

        def pair_body(kk, carry):
            for p in range(2):
                k = 2 * kk + p
                rb = rb0 + k * CHUNK
                sbuf, dbuf = sbufs[p], dbufs[p]
                pltpu.make_async_copy(
                    srcs2.at[pl.ds(rb, CHUNK)], sbuf, isems[p]).wait()
                pltpu.make_async_copy(
                    dsts2.at[pl.ds(rb, CHUNK)], dbuf, isems[p]).wait()

                @pl.when(k + 1 < n_ch)
                def _prefetch():
                    rbn = rb0 + (k + 1) * CHUNK
                    pltpu.async_copy(
                        srcs2.at[pl.ds(rbn, CHUNK)], sbufs[1 - p], isems[1 - p])
                    pltpu.async_copy(
                        dsts2.at[pl.ds(rbn, CHUNK)], dbufs[1 - p], isems[1 - p])

                for g in range(DEPTH):
                    pltpu.async_copy(tbl.at[sbuf.at[g]], rows[g], gsems[g])
                for g in range(CHUNK):
                    b = g % NBUF
                    pltpu.make_async_copy(
                        tbl.at[sbuf.at[g]], rows[b], gsems[b]).wait()
                    pltpu.async_copy(rows[b], acc.at[dbuf.at[g]], ssems[b],
                                     add=True)
                    ng = g + DEPTH
                    if ng < CHUNK:
                        bn = ng % NBUF
                        if ng >= NBUF:
                            pltpu.make_async_copy(
                                rows[bn], acc.at[dbuf.at[ng - NBUF]],
                                ssems[bn]).wait()
                        pltpu.async_copy(tbl.at[sbuf.at[ng]], rows[bn],
                                         gsems[bn])
                for g in range(CHUNK - NBUF, CHUNK):
                    b = g % NBUF
                    pltpu.make_async_copy(
                        rows[b], acc.at[dbuf.at[g]], ssems[b]).wait()
            return carry

        for cc in range(nch):
            pltpu.sync_copy(table.at[cc, pl.ds(r0, STRIPE)],
                            acc.at[pl.ds(r0, STRIPE)])
            pltpu.sync_copy(table.at[cc, pl.ds(r0, STRIPE)],
                            tbl.at[pl.ds(r0, STRIPE)])
            pltpu.async_copy(srcs2.at[pl.ds(rb0, CHUNK)], sbufs[0], isems[0])
            pltpu.async_copy(dsts2.at[pl.ds(rb0, CHUNK)], dbufs[0], isems[0])
            plsc.subcore_barrier()
            lax.fori_loop(0, n_ch // 2, pair_body, 0)
            plsc.subcore_barrier()
            pltpu.sync_copy(acc.at[pl.ds(r0, STRIPE)],
                            out.at[cc, c, pl.ds(r0, STRIPE)])

    return prop


_prop16 = _make_prop(16, 1)
_prop1 = _make_prop(1, 1)


def _blk(*shape):
    return pl.BlockSpec(shape, lambda i: (0,) * (len(shape) - 2) + (i, 0))


def _full(*shape):
    return pl.BlockSpec(shape, lambda i: (0,) * len(shape))


def _stage1_body(degp, xpad, dinv_o, *xs_o):
    deg = degp[0, 0] + degp[0, 1] - 1.0
    row = (pl.program_id(0) * ROWS_BLK
           + lax.broadcasted_iota(jnp.int32, (ROWS_BLK, 1), 0))
    dinv = jnp.where(row < N_REAL, lax.rsqrt(deg), 0.0)
    dinv_o[...] = dinv
    xs = dinv * xpad[...]
    for i in range(NCH1):
        xs_o[i][...] = xs[:, 16 * i:16 * (i + 1)]


def _stage1(degp, xpad):
    return pl.pallas_call(
        _stage1_body,
        grid=(GRID_ROWS,),
        in_specs=[_blk(1, 2, ROWS_BLK, 1), _blk(ROWS_BLK, 48)],
        out_specs=(_blk(ROWS_BLK, 1),) + (_blk(ROWS_BLK, 16),) * NCH1,
        out_shape=(jax.ShapeDtypeStruct((N_PAD, 1), jnp.float32),)
        + (jax.ShapeDtypeStruct((N_PAD, 16), jnp.float32),) * NCH1,
    )(degp, xpad)


def _stage2_body(*refs):
    z1 = refs[:NCH1]
    xs = refs[NCH1:2 * NCH1]
    dinv, w1, b1, w2p = refs[2 * NCH1:2 * NCH1 + 4]
    t2_o = refs[2 * NCH1 + 4:]
    d = dinv[...]
    h1 = b1[...]
    for i in range(NCH1):
        zn = d * (z1[i][0, 0] + z1[i][0, 1] - xs[i][...])
        h1 = h1 + jnp.dot(zn, w1[16 * i:16 * (i + 1), :],
                          preferred_element_type=jnp.float32)
    h1 = jnp.maximum(h1, 0.0)
    t2 = d * jnp.dot(h1, w2p[...], preferred_element_type=jnp.float32)
    for i in range(NCH2):
        t2_o[i][...] = t2[:, 16 * i:16 * (i + 1)]


def _stage2(z1s, xss, dinv, w1, b1r, w2p):
    return pl.pallas_call(
        _stage2_body,
        grid=(GRID_ROWS,),
        in_specs=[_blk(1, 2, ROWS_BLK, 16)] * NCH1 + [_blk(ROWS_BLK, 16)] * NCH1
        + [_blk(ROWS_BLK, 1), _full(48, 300), _full(1, 300), _full(300, 112)],
        out_specs=(_blk(ROWS_BLK, 16),) * NCH2,
        out_shape=(jax.ShapeDtypeStruct((N_PAD, 16), jnp.float32),) * NCH2,
    )(*z1s, *xss, dinv, w1, b1r, w2p)


def _stage3_body(*refs):
    z2 = refs[:NCH2]
    t2 = refs[NCH2:2 * NCH2]
    dinv, b2p, w3p = refs[2 * NCH2:2 * NCH2 + 3]
    t3_o = refs[-1]
    d = dinv[...]
    acc = jnp.zeros((ROWS_BLK, 1), jnp.float32)
    for i in range(NCH2):
        h = jnp.maximum(
            d * (z2[i][0, 0] + z2[i][0, 1] - t2[i][...]) + b2p[:, 16 * i:16 * (i + 1)],
            0.0)
        acc = acc + jnp.dot(h, w3p[16 * i:16 * (i + 1), :],
                            preferred_element_type=jnp.float32)
    t3_o[...] = d * acc


def _stage3(z2s, t2s, dinv, b2p, w3p):
    return pl.pallas_call(
        _stage3_body,
        grid=(GRID_ROWS,),
        in_specs=[_blk(1, 2, ROWS_BLK, 16)] * NCH2 + [_blk(ROWS_BLK, 16)] * NCH2
        + [_blk(ROWS_BLK, 1), _full(1, 112), _full(112, 1)],
        out_specs=_blk(ROWS_BLK, 1),
        out_shape=jax.ShapeDtypeStruct((N_PAD, 1), jnp.float32),
    )(*z2s, *t2s, dinv, b2p, w3p)


def _stage4_body(z3, t3, dinv, b3, out_o):
    out_o[...] = dinv[...] * (z3[0, 0] + z3[0, 1] - t3[...]) + b3[0, 0]


def _stage4(z3, t3, dinv, b3):
    return pl.pallas_call(
        _stage4_body,
        grid=(GRID_ROWS,),
        in_specs=[_blk(1, 2, ROWS_BLK, 1), _blk(ROWS_BLK, 1), _blk(ROWS_BLK, 1),
                  _full(1, 1)],
        out_specs=_blk(ROWS_BLK, 1),
        out_shape=jax.ShapeDtypeStruct((N_PAD, 1), jnp.float32),
    )(z3, t3, dinv, b3)


def kernel(x, edge_index, W1, b1, W2, b2, W3, b3):
    pad = jnp.full((E_PAD - E_REAL,), N_REAL, dtype=jnp.int32)
    srcs = jnp.concatenate([edge_index[0], pad]).reshape(E_PAD // BATCH, BATCH)
    dsts = jnp.concatenate([edge_index[1], pad]).reshape(E_PAD // BATCH, BATCH)
    xpad = jnp.pad(x, ((0, N_PAD - N_REAL), (0, 0)))
    ones = jnp.ones((N_PAD, 1), jnp.float32)
    b1r = b1.reshape(1, 300)
    w2p = jnp.pad(W2, ((0, 0), (0, 12)))
    b2p = jnp.pad(b2, (0, 12)).reshape(1, 112)
    w3p = jnp.pad(W3, ((0, 12), (0, 0)))
    b3r = b3.reshape(1, 1)

    degp = _prop1(ones[None], srcs, dsts)
    s1 = _stage1(degp, xpad)
    dinv, xss = s1[0], s1[1:]
    z1s = [_prop16(t[None], srcs, dsts) for t in xss]
    t2s = _stage2(z1s, xss, dinv, W1, b1r, w2p)
    z2s = [_prop16(t[None], srcs, dsts) for t in t2s]
    t3 = _stage3(z2s, t2s, dinv, b2p, w3p)
    z3 = _prop1(t3[None], srcs, dsts)
    out = _stage4(z3, t3, dinv, b3r)
    return out[:N_REAL, 0]
